# baseline (speedup 1.0000x reference)
import jax
import jax.numpy as jnp
from jax import lax
from jax.experimental import pallas as pl
from jax.experimental.pallas import tpu as pltpu
from jax.experimental.pallas import tpu_sc as plsc

_IGNORE_INDEX = 10

_NC = 2
_NS = 16
_NW = _NC * _NS
_L = 16

_CB = 8
_NQ = 4
_MAXK = 16


def _make_sc_kernel(N, C):
    NB = N // _NQ
    n_tile_rows = C // _CB
    assert C % _CB == 0 and N % (_NQ * 128) == 0 and NB % _L == 0
    n_cg = NB // _L

    mesh = plsc.VectorSubcoreMesh(core_axis_name="c", subcore_axis_name="s")

    @pl.kernel(
        mesh=mesh,
        compiler_params=pltpu.CompilerParams(use_tc_tiling_on_sc=True),
        out_type=jax.ShapeDtypeStruct((C, N), jnp.float32),
        scratch_types=[
            pltpu.VMEM((_CB, NB), jnp.float32),
            pltpu.VMEM((_CB, NB), jnp.float32),
            pltpu.VMEM((NB,), jnp.int32),
            pltpu.VMEM((NB,), jnp.float32),
            pltpu.SemaphoreType.DMA,
            pltpu.SemaphoreType.DMA,
        ],
    )
    def kern(tgt_hbm, grd_hbm, out_hbm, buf0, buf1, tgt_v, grd_v,
             sem0, sem1):
        worker = lax.axis_index("s") * _NC + lax.axis_index("c")
        q = worker % _NQ
        jgrp = worker // _NQ
        col_base = q * NB

        pltpu.sync_copy(tgt_hbm.at[pl.ds(col_base, NB)], tgt_v)
        pltpu.sync_copy(grd_hbm.at[pl.ds(col_base, NB)], grd_v)

        zeros16 = jnp.zeros((_L,), jnp.float32)
        bufs = (buf0, buf1)
        sems = (sem0, sem1)

        def build_and_send(k, buf, sem):
            tile_row = jgrp + 8 * k
            j0 = tile_row * _CB

            @pl.when(tile_row < n_tile_rows)
            def _():
                @pl.when(k >= 2)
                def _():
                    pltpu.make_async_copy(
                        buf,
                        out_hbm.at[pl.ds((jgrp + 8 * (k - 2)) * _CB, _CB),
                                   pl.ds(col_base, NB)],
                        sem).wait()

                def cg_step(cg, carry):
                    base = cg * (4 * _L)
                    for u in range(4):
                        off = base + u * _L
                        d = tgt_v[pl.ds(off, _L)] - j0
                        g = grd_v[pl.ds(off, _L)]
                        for s in range(_CB):
                            buf[s, pl.ds(off, _L)] = jnp.where(
                                d == s, g, zeros16)
                    return carry

                lax.fori_loop(0, n_cg // 4, cg_step, 0)
                pltpu.async_copy(
                    buf,
                    out_hbm.at[pl.ds(j0, _CB), pl.ds(col_base, NB)],
                    sem)

        def step(k, carry):
            @pl.when(k % 2 == 0)
            def _():
                build_and_send(k, buf0, sem0)

            @pl.when(k % 2 == 1)
            def _():
                build_and_send(k, buf1, sem1)

            return carry

        lax.fori_loop(0, _MAXK, step, 0)

        def drain(k, buf, sem):
            tile_row = jgrp + 8 * k

            @pl.when(tile_row < n_tile_rows)
            def _():
                pltpu.make_async_copy(
                    buf,
                    out_hbm.at[pl.ds(tile_row * _CB, _CB),
                               pl.ds(col_base, NB)],
                    sem).wait()

        drain(_MAXK - 2, buf0, sem0)
        drain(_MAXK - 1, buf1, sem1)

    return kern


def kernel(grad_output, input, target, total_weight):
    N, C = input.shape
    t = target.astype(jnp.int32)
    g = grad_output.astype(jnp.float32)
    g2 = jnp.where(t != _IGNORE_INDEX, -g, jnp.zeros_like(g))
    out_t = _make_sc_kernel(N, C)(t, g2)
    return out_t.T


# speedup vs baseline: 1.0409x; 1.0409x over previous
import jax
import jax.numpy as jnp
from jax import lax
from jax.experimental import pallas as pl
from jax.experimental.pallas import tpu as pltpu
from jax.experimental.pallas import tpu_sc as plsc

_IGNORE_INDEX = 10

_NC = 2
_NS = 16
_NW = _NC * _NS
_L = 16

_CB = 8
_NQ = 4
_MAXK = 16


def _make_sc_kernel(N, C):
    NB = N // _NQ
    n_tile_rows = C // _CB
    assert C % _CB == 0 and N % (_NQ * 128) == 0 and NB % _L == 0
    n_cg = NB // _L

    mesh = plsc.VectorSubcoreMesh(core_axis_name="c", subcore_axis_name="s")

    @pl.kernel(
        mesh=mesh,
        compiler_params=pltpu.CompilerParams(use_tc_tiling_on_sc=True),
        out_type=jax.ShapeDtypeStruct((C, N), jnp.float32),
        scratch_types=[
            pltpu.VMEM((_CB, NB), jnp.float32),
            pltpu.VMEM((_CB, NB), jnp.float32),
            pltpu.VMEM((NB,), jnp.int32),
            pltpu.VMEM((NB,), jnp.float32),
            pltpu.SemaphoreType.DMA,
            pltpu.SemaphoreType.DMA,
        ],
    )
    def kern(tgt_hbm, grd_hbm, out_hbm, buf0, buf1, tgt_v, grd_v,
             sem0, sem1):
        worker = lax.axis_index("s") * _NC + lax.axis_index("c")
        q = worker % _NQ
        jgrp = worker // _NQ
        col_base = q * NB

        pltpu.sync_copy(tgt_hbm.at[pl.ds(col_base, NB)], tgt_v)
        pltpu.sync_copy(grd_hbm.at[pl.ds(col_base, NB)], grd_v)

        zeros16 = jnp.zeros((_L,), jnp.float32)
        bufs = (buf0, buf1)
        sems = (sem0, sem1)

        def build_and_send(k, buf, sem):
            tile_row = jgrp + 8 * k
            j0 = tile_row * _CB

            @pl.when(tile_row < n_tile_rows)
            def _():
                @pl.when(k >= 2)
                def _():
                    pltpu.make_async_copy(
                        buf,
                        out_hbm.at[pl.ds((jgrp + 8 * (k - 2)) * _CB, _CB),
                                   pl.ds(col_base, NB)],
                        sem).wait()

                def cg_step(cg, carry):
                    base = cg * (4 * _L)
                    for u in range(4):
                        off = base + u * _L
                        t = tgt_v[pl.ds(off, _L)]
                        g = grd_v[pl.ds(off, _L)]
                        for s in range(_CB):
                            buf[s, pl.ds(off, _L)] = jnp.where(
                                t == j0 + s, g, zeros16)
                    return carry

                lax.fori_loop(0, n_cg // 4, cg_step, 0)
                pltpu.async_copy(
                    buf,
                    out_hbm.at[pl.ds(j0, _CB), pl.ds(col_base, NB)],
                    sem)

        def step(k, carry):
            @pl.when(k % 2 == 0)
            def _():
                build_and_send(k, buf0, sem0)

            @pl.when(k % 2 == 1)
            def _():
                build_and_send(k, buf1, sem1)

            return carry

        lax.fori_loop(0, _MAXK, step, 0)

        def drain(k, buf, sem):
            tile_row = jgrp + 8 * k

            @pl.when(tile_row < n_tile_rows)
            def _():
                pltpu.make_async_copy(
                    buf,
                    out_hbm.at[pl.ds(tile_row * _CB, _CB),
                               pl.ds(col_base, NB)],
                    sem).wait()

        drain(_MAXK - 2, buf0, sem0)
        drain(_MAXK - 1, buf1, sem1)

    return kern


def kernel(grad_output, input, target, total_weight):
    N, C = input.shape
    t = target.astype(jnp.int32)
    g = grad_output.astype(jnp.float32)
    g2 = jnp.where(t != _IGNORE_INDEX, -g, jnp.zeros_like(g))
    out_t = _make_sc_kernel(N, C)(t, g2)
    return out_t.T
